# baseline (speedup 1.0000x reference)
import functools

import jax
import jax.numpy as jnp
from jax import lax
from jax.experimental import pallas as pl
from jax.experimental.pallas import tpu as pltpu
from jax.experimental.pallas import tpu_sc as plsc

SB = 16384


def _argmin_body(n2, q2t_ref, ob_ref, k_ref, out_ref, best_val, best_idx):
    pid = pl.program_id(0)
    nb = pl.num_programs(0)

    @pl.when(pid == 0)
    def _init():
        best_val[...] = jnp.full_like(best_val, jnp.inf)
        best_idx[...] = jnp.zeros_like(best_idx)

    kb = k_ref[...]
    s = lax.dot_general(kb, q2t_ref[...], (((1,), (0,)), ((), ())),
                        preferred_element_type=jnp.float32)
    s = s + lax.dot_general(kb * kb, ob_ref[...], (((1,), (0,)), ((), ())),
                            preferred_element_type=jnp.float32)
    ridx = lax.broadcasted_iota(jnp.int32, s.shape, 0)
    big = jnp.int32(jnp.iinfo(jnp.int32).max)

    def _update(s):
        local_min = jnp.min(s, axis=0, keepdims=True)
        local_arg = jnp.min(jnp.where(s == local_min, ridx, big),
                            axis=0, keepdims=True)
        improved = local_min < best_val[...]
        best_val[...] = jnp.where(improved, local_min, best_val[...])
        best_idx[...] = jnp.where(improved, pid * SB + local_arg,
                                  best_idx[...])

    @pl.when(pid < nb - 1)
    def _main():
        _update(s)

    @pl.when(pid == nb - 1)
    def _fin():
        _update(jnp.where(ridx < (n2 - pid * SB), s, jnp.inf))
        bi = best_idx[...]
        bv = best_val[...]
        ke = 2 * bi[:, :64]
        ko = 2 * bi[:, 64:] + 1
        ve = bv[:, :64]
        vo = bv[:, 64:]
        out_ref[...] = jnp.where(
            ve < vo, ke, jnp.where(vo < ve, ko, jnp.minimum(ke, ko)))


def _nearest_tc(queries, keys):
    q, d = queries.shape
    n2 = keys.shape[0] // 2
    kv = keys.reshape(n2, 2 * d)
    qm = jnp.transpose(queries) * -2.0
    z = jnp.zeros((d, q), jnp.float32)
    q2t = jnp.concatenate(
        [jnp.concatenate([qm, z], axis=1),
         jnp.concatenate([z, qm], axis=1)], axis=0)
    o64 = jnp.ones((d, q), jnp.float32)
    ob = jnp.concatenate(
        [jnp.concatenate([o64, z], axis=1),
         jnp.concatenate([z, o64], axis=1)], axis=0)
    grid = (n2 + SB - 1) // SB
    return pl.pallas_call(
        functools.partial(_argmin_body, n2),
        grid=(grid,),
        in_specs=[
            pl.BlockSpec((2 * d, 2 * q), lambda i: (0, 0)),
            pl.BlockSpec((2 * d, 2 * q), lambda i: (0, 0)),
            pl.BlockSpec((SB, 2 * d), lambda i: (i, 0)),
        ],
        out_specs=pl.BlockSpec((1, q), lambda i: (0, 0)),
        out_shape=jax.ShapeDtypeStruct((1, q), jnp.int32),
        scratch_shapes=[
            pltpu.VMEM((1, 2 * q), jnp.float32),
            pltpu.VMEM((1, 2 * q), jnp.int32),
        ],
    )(q2t, ob, kv)


def _sc_chase_body(nearest_hbm, state_hbm, sidx_out, idx_v, sidx_v, sem):
    wid = lax.axis_index("s") * 2 + lax.axis_index("c")

    @pl.when(wid == 0)
    def _():
        pltpu.sync_copy(nearest_hbm, idx_v)
        pltpu.async_copy(state_hbm.at[idx_v], sidx_v, sem).wait()
        pltpu.sync_copy(sidx_v, sidx_out)


def _chase_sc(nearest, state_indices):
    q = nearest.shape[0]
    mesh = plsc.VectorSubcoreMesh(core_axis_name="c", subcore_axis_name="s")
    return pl.kernel(
        _sc_chase_body,
        out_type=jax.ShapeDtypeStruct((q,), jnp.int32),
        mesh=mesh,
        scratch_types=[
            pltpu.VMEM((q,), jnp.int32),
            pltpu.VMEM((q,), jnp.int32),
            pltpu.SemaphoreType.DMA,
        ],
    )(nearest, state_indices)


def _row_gather_body(sidx_ref, cc_ref, out_ref):
    i = pl.program_id(0)
    r = sidx_ref[i] % 8
    out_ref[pl.ds(i % 8, 1), :] = cc_ref[pl.ds(r, 1), :]


def _gather_tc(cluster_centers, sidx):
    q = sidx.shape[0]
    dc = cluster_centers.shape[1]
    grid_spec = pltpu.PrefetchScalarGridSpec(
        num_scalar_prefetch=1,
        grid=(q,),
        in_specs=[pl.BlockSpec((8, dc), lambda i, s: (s[i] // 8, 0))],
        out_specs=pl.BlockSpec((8, dc), lambda i, s: (i // 8, 0)),
    )
    return pl.pallas_call(
        _row_gather_body,
        grid_spec=grid_spec,
        out_shape=jax.ShapeDtypeStruct((q, dc), jnp.float32),
    )(sidx, cluster_centers)


def kernel(queries, keys, cluster_centers, state_indices):
    q = queries.shape[0]
    nearest = _nearest_tc(queries, keys).reshape((q,))
    sidx = _chase_sc(nearest, state_indices)
    return _gather_tc(cluster_centers, sidx)
